# Initial kernel scaffold; baseline (speedup 1.0000x reference)
import jax
import jax.numpy as jnp
from jax.experimental import pallas as pl


def kernel(y_pred, y_true):
    raise NotImplementedError("write your pallas kernel here")


# speedup vs baseline: 3960.3485x; 3960.3485x over previous
import functools

import jax
import jax.numpy as jnp
from jax import lax
from jax.experimental import pallas as pl
from jax.experimental.pallas import tpu as pltpu
from jax.experimental.pallas import tpu_sc as plsc

_BINS = 50
_EPS = 1e-8
_B = 32
_N = 512 * 512
_NC = 2
_NS = 16
_NW = _NC * _NS
_CH = 32768
_NCHUNK = _N // _CH


def _stage1_body(yp_ref, yt_ref, idx_ref, mm_ref):
    yp = yp_ref[0]
    p = jnp.maximum(jnp.sqrt(yp[0] * yp[0] + yp[1] * yp[1]), 1e-6)
    yt = yt_ref[0]
    t = jnp.maximum(jnp.sqrt(yt[0] * yt[0] + yt[1] * yt[1]), 1e-6)
    pmin = jnp.min(p)
    pmax = jnp.max(p)
    scale = _BINS / jnp.maximum(pmax - pmin, 1e-30)
    idxp = jnp.clip(jnp.floor((p - pmin) * scale).astype(jnp.int32), 0, _BINS - 1)
    in_t = (t >= pmin) & (t <= pmax)
    idxt = jnp.clip(jnp.floor((t - pmin) * scale).astype(jnp.int32), 0, _BINS - 1)
    idxt = jnp.where(in_t, idxt + 64, 127)
    idx_ref[0] = idxp | (idxt << 8)
    col = lax.broadcasted_iota(jnp.int32, (1, 1, 128), 2)
    mm_ref[...] = jnp.where(col == 0, pmin, jnp.where(col == 1, pmax, 0.0))


def _stage1(y_pred, y_true):
    return pl.pallas_call(
        _stage1_body,
        grid=(_B,),
        in_specs=[
            pl.BlockSpec((1, 2, 512, 512), lambda s: (s, 0, 0, 0)),
            pl.BlockSpec((1, 2, 512, 512), lambda s: (s, 0, 0, 0)),
        ],
        out_specs=[
            pl.BlockSpec((1, 512, 512), lambda s: (s, 0, 0)),
            pl.BlockSpec((1, 1, 128), lambda s: (s, 0, 0)),
        ],
        out_shape=[
            jax.ShapeDtypeStruct((_B, 512, 512), jnp.int32),
            jax.ShapeDtypeStruct((_B, 1, 128), jnp.float32),
        ],
    )(y_pred, y_true)


def _sc_hist_body(idx_hbm, outp_hbm, outt_hbm, buf0, buf1, hist, sem0, sem1):
    wid = lax.axis_index("s") * _NC + lax.axis_index("c")
    zero = jnp.zeros((16,), jnp.float32)

    def zrow(r, carry):
        hist[pl.ds(r * 16, 16)] = zero
        return carry

    lax.fori_loop(0, 128, zrow, 0)

    lane = lax.iota(jnp.int32, 16)
    ones = jnp.ones((16,), jnp.float32)
    sems = (sem0, sem1)
    bufs = (buf0, buf1)

    def src(ci):
        return idx_hbm.at[wid, pl.ds(ci * _CH, _CH)]

    pltpu.async_copy(src(0), bufs[0], sems[0])
    for ci in range(_NCHUNK):
        b = ci % 2
        if ci + 1 < _NCHUNK:
            pltpu.async_copy(src(ci + 1), bufs[(ci + 1) % 2], sems[(ci + 1) % 2])
        pltpu.make_async_copy(src(ci), bufs[b], sems[b]).wait()
        bref = bufs[b]

        def ibody(j, carry):
            v = bref[pl.ds(j * 16, 16)]
            lo = ((v << 4) & 0x7F0) | lane
            hi = (lax.shift_right_logical(v, 4) & 0x7F0) | lane
            plsc.addupdate_scatter(hist, [lo], ones)
            plsc.addupdate_scatter(hist, [hi], ones)
            return carry

        lax.fori_loop(0, _CH // 16, ibody, 0)

    pltpu.sync_copy(hist.at[pl.ds(0, 1024)], outp_hbm.at[wid])
    pltpu.sync_copy(hist.at[pl.ds(1024, 1024)], outt_hbm.at[wid])


def _sc_hist(idx_flat):
    mesh = plsc.VectorSubcoreMesh(core_axis_name="c", subcore_axis_name="s")
    f = pl.kernel(
        _sc_hist_body,
        out_type=[
            jax.ShapeDtypeStruct((_B, 1024), jnp.float32),
            jax.ShapeDtypeStruct((_B, 1024), jnp.float32),
        ],
        mesh=mesh,
        compiler_params=pltpu.CompilerParams(needs_layout_passes=False),
        scratch_types=[
            pltpu.VMEM((_CH,), jnp.int32),
            pltpu.VMEM((_CH,), jnp.int32),
            pltpu.VMEM((2048,), jnp.float32),
            pltpu.SemaphoreType.DMA,
            pltpu.SemaphoreType.DMA,
        ],
    )
    return f(idx_flat)


def _kl_body(cp_ref, ct_ref, mm_ref, out_ref):
    cp = jnp.sum(cp_ref[...], axis=2)
    ct = jnp.sum(ct_ref[...], axis=2)
    mm = mm_ref[...]
    pmin = mm[:, 0:1]
    pmax = mm[:, 1:2]
    valid = lax.broadcasted_iota(jnp.int32, (_B, 64), 1) < _BINS
    cp = jnp.where(valid, cp, 0.0)
    ct = jnp.where(valid, ct, 0.0)
    tot_p = jnp.maximum(jnp.sum(cp, axis=1, keepdims=True), 1.0)
    tot_t = jnp.maximum(jnp.sum(ct, axis=1, keepdims=True), 1.0)
    w = jnp.maximum(pmax - pmin, 1e-30) / _BINS
    hp = jnp.where(valid, cp / (w * tot_p) + _EPS, 0.0)
    ht = jnp.where(valid, ct / (w * tot_t) + _EPS, 0.0)
    hp = hp / jnp.sum(hp, axis=1, keepdims=True)
    ht = ht / jnp.sum(ht, axis=1, keepdims=True)
    ratio = jnp.where(valid, ht / hp, 1.0)
    kl = jnp.sum(jnp.where(valid, ht * jnp.log(ratio), 0.0), axis=1)
    out_ref[...] = jnp.broadcast_to(jnp.sum(kl) / _B, (1, 1))


def _kl(cp, ct, mm):
    return pl.pallas_call(
        _kl_body,
        out_shape=jax.ShapeDtypeStruct((1, 1), jnp.float32),
    )(cp, ct, mm)


def kernel(y_pred, y_true):
    idx, mm = _stage1(y_pred, y_true)
    idx_flat = idx.reshape(_B, _N)
    cp, ct = _sc_hist(idx_flat)
    out = _kl(cp.reshape(_B, 64, 16), ct.reshape(_B, 64, 16), mm.reshape(_B, 128))
    return out.reshape(())


# speedup vs baseline: 6276.8997x; 1.5849x over previous
import functools

import jax
import jax.numpy as jnp
from jax import lax
from jax.experimental import pallas as pl
from jax.experimental.pallas import tpu as pltpu
from jax.experimental.pallas import tpu_sc as plsc

_BINS = 50
_EPS = 1e-8
_B = 32
_N = 512 * 512
_NC = 2
_NS = 16
_NW = _NC * _NS
_CH = 32768
_NCHUNK = _N // _CH


def _stage1_body(yp_ref, yt_ref, idx_ref, mm_ref):
    yp = yp_ref[0]
    p = jnp.maximum(jnp.sqrt(yp[0] * yp[0] + yp[1] * yp[1]), 1e-6)
    yt = yt_ref[0]
    t = jnp.maximum(jnp.sqrt(yt[0] * yt[0] + yt[1] * yt[1]), 1e-6)
    pmin = jnp.min(p)
    pmax = jnp.max(p)
    scale = _BINS / jnp.maximum(pmax - pmin, 1e-30)
    idxp = jnp.clip(jnp.floor((p - pmin) * scale).astype(jnp.int32), 0, _BINS - 1)
    in_t = (t >= pmin) & (t <= pmax)
    idxt = jnp.clip(jnp.floor((t - pmin) * scale).astype(jnp.int32), 0, _BINS - 1)
    idxt = jnp.where(in_t, idxt + 64, 127)
    idx_ref[0] = idxp | (idxt << 8)
    col = lax.broadcasted_iota(jnp.int32, (1, 1, 128), 2)
    mm_ref[...] = jnp.where(col == 0, pmin, jnp.where(col == 1, pmax, 0.0))


def _stage1(y_pred, y_true):
    return pl.pallas_call(
        _stage1_body,
        grid=(_B,),
        in_specs=[
            pl.BlockSpec((1, 2, 512, 512), lambda s: (s, 0, 0, 0)),
            pl.BlockSpec((1, 2, 512, 512), lambda s: (s, 0, 0, 0)),
        ],
        out_specs=[
            pl.BlockSpec((1, 512, 512), lambda s: (s, 0, 0)),
            pl.BlockSpec((1, 1, 128), lambda s: (s, 0, 0)),
        ],
        out_shape=[
            jax.ShapeDtypeStruct((_B, 512, 512), jnp.int32),
            jax.ShapeDtypeStruct((_B, 1, 128), jnp.float32),
        ],
    )(y_pred, y_true)


def _sc_hist_body(idx_hbm, outp_hbm, outt_hbm, buf0, buf1, hist, sem0, sem1):
    wid = lax.axis_index("s") * _NC + lax.axis_index("c")
    zero = jnp.zeros((16,), jnp.float32)

    def zrow(r, carry):
        hist[pl.ds(r * 16, 16)] = zero
        return carry

    lax.fori_loop(0, 128, zrow, 0)

    lane = lax.iota(jnp.int32, 16)
    ones = jnp.ones((16,), jnp.float32)
    sems = (sem0, sem1)
    bufs = (buf0, buf1)

    def src(ci):
        return idx_hbm.at[wid, pl.ds(ci * _CH, _CH)]

    pltpu.async_copy(src(0), bufs[0], sems[0])
    for ci in range(_NCHUNK):
        b = ci % 2
        if ci + 1 < _NCHUNK:
            pltpu.async_copy(src(ci + 1), bufs[(ci + 1) % 2], sems[(ci + 1) % 2])
        pltpu.make_async_copy(src(ci), bufs[b], sems[b]).wait()
        bref = bufs[b]

        @plsc.parallel_loop(0, _CH // 16, unroll=8)
        def ibody(j):
            v = bref[pl.ds(j * 16, 16)]
            lo = ((v << 4) & 0x7F0) | lane
            hi = (lax.shift_right_logical(v, 4) & 0x7F0) | lane
            plsc.addupdate_scatter(hist, [lo], ones)
            plsc.addupdate_scatter(hist, [hi], ones)

    pltpu.sync_copy(hist.at[pl.ds(0, 1024)], outp_hbm.at[wid])
    pltpu.sync_copy(hist.at[pl.ds(1024, 1024)], outt_hbm.at[wid])


def _sc_hist(idx_flat):
    mesh = plsc.VectorSubcoreMesh(core_axis_name="c", subcore_axis_name="s")
    f = pl.kernel(
        _sc_hist_body,
        out_type=[
            jax.ShapeDtypeStruct((_B, 1024), jnp.float32),
            jax.ShapeDtypeStruct((_B, 1024), jnp.float32),
        ],
        mesh=mesh,
        compiler_params=pltpu.CompilerParams(needs_layout_passes=False),
        scratch_types=[
            pltpu.VMEM((_CH,), jnp.int32),
            pltpu.VMEM((_CH,), jnp.int32),
            pltpu.VMEM((2048,), jnp.float32),
            pltpu.SemaphoreType.DMA,
            pltpu.SemaphoreType.DMA,
        ],
    )
    return f(idx_flat)


def _kl_body(cp_ref, ct_ref, mm_ref, out_ref):
    cp = jnp.sum(cp_ref[...], axis=2)
    ct = jnp.sum(ct_ref[...], axis=2)
    mm = mm_ref[...]
    pmin = mm[:, 0:1]
    pmax = mm[:, 1:2]
    valid = lax.broadcasted_iota(jnp.int32, (_B, 64), 1) < _BINS
    cp = jnp.where(valid, cp, 0.0)
    ct = jnp.where(valid, ct, 0.0)
    tot_p = jnp.maximum(jnp.sum(cp, axis=1, keepdims=True), 1.0)
    tot_t = jnp.maximum(jnp.sum(ct, axis=1, keepdims=True), 1.0)
    w = jnp.maximum(pmax - pmin, 1e-30) / _BINS
    hp = jnp.where(valid, cp / (w * tot_p) + _EPS, 0.0)
    ht = jnp.where(valid, ct / (w * tot_t) + _EPS, 0.0)
    hp = hp / jnp.sum(hp, axis=1, keepdims=True)
    ht = ht / jnp.sum(ht, axis=1, keepdims=True)
    ratio = jnp.where(valid, ht / hp, 1.0)
    kl = jnp.sum(jnp.where(valid, ht * jnp.log(ratio), 0.0), axis=1)
    out_ref[...] = jnp.broadcast_to(jnp.sum(kl) / _B, (1, 1))


def _kl(cp, ct, mm):
    return pl.pallas_call(
        _kl_body,
        out_shape=jax.ShapeDtypeStruct((1, 1), jnp.float32),
    )(cp, ct, mm)


def kernel(y_pred, y_true):
    idx, mm = _stage1(y_pred, y_true)
    idx_flat = idx.reshape(_B, _N)
    cp, ct = _sc_hist(idx_flat)
    out = _kl(cp.reshape(_B, 64, 16), ct.reshape(_B, 64, 16), mm.reshape(_B, 128))
    return out.reshape(())
